# baseline (speedup 1.0000x reference)
import functools

import jax
import jax.numpy as jnp
from jax import lax
from jax.experimental import pallas as pl
from jax.experimental.pallas import tpu as pltpu
from jax.experimental.pallas import tpu_sc as plsc

ITEM_NUM = 100000
CATE_NUM = 1000
HIDDEN = 64
B = 1024
L = 50
N_NODES = 30
N_EDGES = 20

_BB = 32
_VB = 2048
_NVB = -(-ITEM_NUM // _VB)


def _sc_gather(emb, idx):
    info = plsc.get_sparse_core_info()
    nw = info.num_cores * info.num_subcores
    n, d = idx.shape[0], emb.shape[1]
    bpw = n // nw
    mesh = plsc.VectorSubcoreMesh(core_axis_name="c", subcore_axis_name="s")

    @functools.partial(
        pl.kernel,
        mesh=mesh,
        out_type=jax.ShapeDtypeStruct((n, d), jnp.float32),
        scratch_types=[
            pltpu.VMEM((bpw,), jnp.int32),
            pltpu.VMEM((bpw, d), jnp.float32),
            pltpu.SemaphoreType.DMA,
        ],
    )
    def gather_k(emb_hbm, idx_hbm, out_hbm, idx_v, rows_v, sem):
        wid = lax.axis_index("s") * info.num_cores + lax.axis_index("c")
        base = wid * bpw
        pltpu.sync_copy(idx_hbm.at[pl.ds(base, bpw)], idx_v)
        pltpu.async_copy(emb_hbm.at[idx_v], rows_v, sem).wait()
        pltpu.sync_copy(rows_v, out_hbm.at[pl.ds(base, bpw)])

    return gather_k(emb, idx)


def _frontend_body(nh_ref, par_ref, adj_ref, ai_ref, ac_ref, seq_ref,
                   w1_ref, w2_ref, ab_ref, av_ref, lo_ref, hi_ref):
    raw = nh_ref[...]
    par = par_ref[...]
    x = jnp.where(par == 1.0,
                  raw[:, :, HIDDEN:], raw[:, :, :HIDDEN])
    n2 = jnp.sum(x * x, axis=2, keepdims=True)
    x = x / jnp.maximum(jnp.sqrt(n2), 1e-12)
    h = adj_ref[...]
    dv3 = jnp.maximum(jnp.sum(h, axis=2, keepdims=True), 1e-6)
    de3 = jnp.maximum(jnp.sum(h, axis=1, keepdims=True), 1e-6)

    edges = []
    for m in range(N_EDGES):
        e = jnp.sum(h[:, :, m:m + 1] * x, axis=1, keepdims=True)
        edges.append(e / de3[:, :, m:m + 1])

    x2 = []
    for n in range(N_NODES):
        acc = h[:, n:n + 1, 0:1] * edges[0]
        for m in range(1, N_EDGES):
            acc = acc + h[:, n:n + 1, m:m + 1] * edges[m]
        x2.append(acc / dv3[:, n:n + 1, :])

    ai = ai_ref[...]
    ac = ac_ref[...]
    s_lo = jnp.zeros((_BB, L, HIDDEN), jnp.float32)
    s_hi = jnp.zeros((_BB, L, HIDDEN), jnp.float32)
    for n in range(N_NODES):
        s_lo = jnp.where(ai == n, x2[n], s_lo)
        s_hi = jnp.where(ac == n, x2[n], s_hi)

    nrm = jnp.sqrt(jnp.sum(s_lo * s_lo, axis=2, keepdims=True)
                   + jnp.sum(s_hi * s_hi, axis=2, keepdims=True))
    inv = 1.0 / jnp.maximum(nrm, 1e-12)
    s_lo = s_lo * inv
    s_hi = s_hi * inv

    mask3 = (seq_ref[...] > 0)
    maskf3 = mask3.astype(jnp.float32)
    cnt3 = jnp.sum(mask3.astype(jnp.int32), axis=1, keepdims=True)
    last3 = jnp.clip(cnt3 - 1, 0, L - 1)
    lidx3 = lax.broadcasted_iota(jnp.int32, (_BB, L, 1), 1)
    sel3 = (lidx3 == last3).astype(jnp.float32)
    ht_lo = jnp.sum(s_lo * sel3, axis=1)
    ht_hi = jnp.sum(s_hi * sel3, axis=1)

    w1 = w1_ref[...]
    w2 = w2_ref[...]
    h1 = (jnp.dot(ht_lo, w1[:HIDDEN, :], preferred_element_type=jnp.float32)
          + jnp.dot(ht_hi, w1[HIDDEN:, :], preferred_element_type=jnp.float32))
    f_lo = s_lo.reshape(_BB * L, HIDDEN)
    f_hi = s_hi.reshape(_BB * L, HIDDEN)
    s2 = (jnp.dot(f_lo, w2[:HIDDEN, :], preferred_element_type=jnp.float32)
          + jnp.dot(f_hi, w2[HIDDEN:, :], preferred_element_type=jnp.float32))
    a = s2.reshape(_BB, L, 2 * HIDDEN) + h1[:, None, :] + ab_ref[...][None, :, :]
    a = jax.nn.sigmoid(a)
    alpha3 = jnp.sum(a * av_ref[...][None, :, :], axis=2,
                     keepdims=True) * maskf3

    out_lo = jnp.sum(alpha3 * s_lo, axis=1)
    out_hi = jnp.sum(alpha3 * s_hi, axis=1)
    on = jnp.sqrt(jnp.sum(out_lo * out_lo, axis=1, keepdims=True)
                  + jnp.sum(out_hi * out_hi, axis=1, keepdims=True))
    oinv = 1.0 / jnp.maximum(on, 1e-12)
    lo_ref[...] = out_lo * oinv
    hi_ref[...] = out_hi * oinv


def _frontend(nh, par, hn_adj, alias_item, alias_cate, item_seq, w1, w2, att_b, att_v):
    nb = B // _BB
    grid = (nb,)
    bs = [
        pl.BlockSpec((_BB, N_NODES, 2 * HIDDEN), lambda i: (i, 0, 0)),
        pl.BlockSpec((_BB, N_NODES, 1), lambda i: (i, 0, 0)),
        pl.BlockSpec((_BB, N_NODES, N_EDGES), lambda i: (i, 0, 0)),
        pl.BlockSpec((_BB, L, 1), lambda i: (i, 0, 0)),
        pl.BlockSpec((_BB, L, 1), lambda i: (i, 0, 0)),
        pl.BlockSpec((_BB, L, 1), lambda i: (i, 0, 0)),
        pl.BlockSpec((2 * HIDDEN, 2 * HIDDEN), lambda i: (0, 0)),
        pl.BlockSpec((2 * HIDDEN, 2 * HIDDEN), lambda i: (0, 0)),
        pl.BlockSpec((1, 2 * HIDDEN), lambda i: (0, 0)),
        pl.BlockSpec((1, 2 * HIDDEN), lambda i: (0, 0)),
    ]
    out_shape = [
        jax.ShapeDtypeStruct((B, HIDDEN), jnp.float32),
        jax.ShapeDtypeStruct((B, HIDDEN), jnp.float32),
    ]
    out_bs = [
        pl.BlockSpec((_BB, HIDDEN), lambda i: (i, 0)),
        pl.BlockSpec((_BB, HIDDEN), lambda i: (i, 0)),
    ]
    return pl.pallas_call(
        _frontend_body, grid=grid, in_specs=bs, out_specs=out_bs,
        out_shape=out_shape,
    )(nh, par, hn_adj, alias_item, alias_cate, item_seq, w1, w2,
      att_b.reshape(1, -1), att_v.reshape(1, -1))


_CPAD = 1024


def _scores_body(lo_ref, hi_ref, emb_ref, cate_ref, ctbl_ref, out_ref):
    cb = cate_ref[...].reshape(1, _VB) - ITEM_NUM
    biota = lax.broadcasted_iota(jnp.int32, (128, _VB), 0)
    ctblt = ctbl_ref[...]
    chalf = None
    for j in range(_CPAD // 128):
        oh = (biota + (j * 128) == cb).astype(jnp.float32)
        p = jnp.dot(ctblt[:, j * 128:(j + 1) * 128], oh,
                    preferred_element_type=jnp.float32)
        chalf = p if chalf is None else chalf + p

    eb = emb_ref[...]
    ones = jnp.ones((1, HIDDEN), jnp.float32)
    n2 = (lax.dot_general(ones, eb * eb,
                          (((1,), (1,)), ((), ())),
                          preferred_element_type=jnp.float32)
          + jnp.dot(ones, chalf * chalf,
                    preferred_element_type=jnp.float32))
    scale = 16.0 / jnp.maximum(jnp.sqrt(n2), 1e-12)

    s = (lax.dot_general(lo_ref[...], eb, (((1,), (1,)), ((), ())),
                         preferred_element_type=jnp.float32)
         + jnp.dot(hi_ref[...], chalf, preferred_element_type=jnp.float32))
    out_ref[...] = s * scale


def _scores(seq_lo, seq_hi, emb_items, cates3, ctblt):
    nv = _NVB
    bs = [
        pl.BlockSpec((B, HIDDEN), lambda v: (0, 0)),
        pl.BlockSpec((B, HIDDEN), lambda v: (0, 0)),
        pl.BlockSpec((_VB, HIDDEN), lambda v: (v, 0)),
        pl.BlockSpec((1, 1, _VB), lambda v: (v, 0, 0)),
        pl.BlockSpec((HIDDEN, _CPAD), lambda v: (0, 0)),
    ]
    return pl.pallas_call(
        _scores_body, grid=(nv,), in_specs=bs,
        out_specs=pl.BlockSpec((B, _VB), lambda v: (0, v)),
        out_shape=jax.ShapeDtypeStruct((B, ITEM_NUM), jnp.float32),
    )(seq_lo, seq_hi, emb_items, cates3, ctblt)


def kernel(item_seq, label, nodes, hn_adj, alias_item, alias_cate, emb,
           item_cates, W1, W2, att_b, att_v):
    del label
    emb = emb.astype(jnp.float32)
    nodes_flat = nodes.reshape(-1).astype(jnp.int32)
    emb_pairs = emb.reshape((ITEM_NUM + CATE_NUM) // 2, 2 * HIDDEN)
    nh = _sc_gather(emb_pairs, nodes_flat >> 1).reshape(B, N_NODES, 2 * HIDDEN)
    par = (nodes_flat & 1).astype(jnp.float32).reshape(B, N_NODES, 1)

    seq_lo, seq_hi = _frontend(
        nh, par, hn_adj.astype(jnp.float32),
        alias_item.astype(jnp.int32).reshape(B, L, 1),
        alias_cate.astype(jnp.int32).reshape(B, L, 1),
        item_seq.astype(jnp.int32).reshape(B, L, 1),
        W1.astype(jnp.float32), W2.astype(jnp.float32),
        att_b.astype(jnp.float32), att_v.astype(jnp.float32))

    emb_items = emb[:ITEM_NUM]
    ctblt = jnp.pad(emb[ITEM_NUM:].T, ((0, 0), (0, _CPAD - CATE_NUM)))
    cates_pad = jnp.pad(item_cates.astype(jnp.int32),
                        (0, _NVB * _VB - ITEM_NUM),
                        constant_values=ITEM_NUM)
    cates3 = cates_pad.reshape(_NVB, 1, _VB)
    return _scores(seq_lo, seq_hi, emb_items, cates3, ctblt)


# speedup vs baseline: 1.8241x; 1.8241x over previous
import functools

import jax
import jax.numpy as jnp
from jax import lax
from jax.experimental import pallas as pl
from jax.experimental.pallas import tpu as pltpu
from jax.experimental.pallas import tpu_sc as plsc

ITEM_NUM = 100000
CATE_NUM = 1000
HIDDEN = 64
B = 1024
L = 50
N_NODES = 30
N_EDGES = 20

_BL = 128
_VB = 2048
_NVB = -(-ITEM_NUM // _VB)


def _sc_gather(emb, idx):
    info = plsc.get_sparse_core_info()
    nw = info.num_cores * info.num_subcores
    n, d = idx.shape[0], emb.shape[1]
    bpw = n // nw
    mesh = plsc.VectorSubcoreMesh(core_axis_name="c", subcore_axis_name="s")

    @functools.partial(
        pl.kernel,
        mesh=mesh,
        out_type=jax.ShapeDtypeStruct((n, d), jnp.float32),
        scratch_types=[
            pltpu.VMEM((bpw,), jnp.int32),
            pltpu.VMEM((bpw, d), jnp.float32),
            pltpu.SemaphoreType.DMA,
        ],
    )
    def gather_k(emb_hbm, idx_hbm, out_hbm, idx_v, rows_v, sem):
        wid = lax.axis_index("s") * info.num_cores + lax.axis_index("c")
        base = wid * bpw
        pltpu.sync_copy(idx_hbm.at[pl.ds(base, bpw)], idx_v)
        pltpu.async_copy(emb_hbm.at[idx_v], rows_v, sem).wait()
        pltpu.sync_copy(rows_v, out_hbm.at[pl.ds(base, bpw)])

    return gather_k(emb, idx)


def _frontend_body(nh_ref, par_ref, adj_ref, ai_ref, ac_ref, seq_ref,
                   w1_ref, w2_ref, ab_ref, av_ref, lo_ref, hi_ref):
    raw = nh_ref[...]
    par = par_ref[...]
    x = jnp.where(par == 1.0, raw[:, HIDDEN:, :], raw[:, :HIDDEN, :])
    n2 = jnp.sum(x * x, axis=1, keepdims=True)
    x = x / jnp.maximum(jnp.sqrt(n2), 1e-12)

    adj = adj_ref[...]
    dv = jnp.maximum(jnp.sum(adj, axis=1, keepdims=True), 1e-6)
    de = adj[0]
    for n in range(1, N_NODES):
        de = de + adj[n]
    de = jnp.maximum(de, 1e-6)

    edges = []
    for m in range(N_EDGES):
        e = adj[0, m:m + 1, :] * x[0]
        for n in range(1, N_NODES):
            e = e + adj[n, m:m + 1, :] * x[n]
        edges.append(e / de[m:m + 1, :])

    ai = ai_ref[...]
    ac = ac_ref[...]
    ha_i = jnp.zeros((L, N_EDGES, _BL), jnp.float32)
    ha_c = jnp.zeros((L, N_EDGES, _BL), jnp.float32)
    dva_i = jnp.zeros((L, 1, _BL), jnp.float32)
    dva_c = jnp.zeros((L, 1, _BL), jnp.float32)
    for n in range(N_NODES):
        mi = (ai == n).astype(jnp.float32)
        mc = (ac == n).astype(jnp.float32)
        an = adj[n][None, :, :]
        ha_i = ha_i + mi * an
        ha_c = ha_c + mc * an
        dva_i = dva_i + mi * dv[n][None, :, :]
        dva_c = dva_c + mc * dv[n][None, :, :]

    s_lo = jnp.zeros((L, HIDDEN, _BL), jnp.float32)
    s_hi = jnp.zeros((L, HIDDEN, _BL), jnp.float32)
    for m in range(N_EDGES):
        em = edges[m][None, :, :]
        s_lo = s_lo + ha_i[:, m:m + 1, :] * em
        s_hi = s_hi + ha_c[:, m:m + 1, :] * em
    s_lo = s_lo / dva_i
    s_hi = s_hi / dva_c

    nrm = jnp.sqrt(jnp.sum(s_lo * s_lo, axis=1, keepdims=True)
                   + jnp.sum(s_hi * s_hi, axis=1, keepdims=True))
    inv = 1.0 / jnp.maximum(nrm, 1e-12)
    s_lo = s_lo * inv
    s_hi = s_hi * inv

    maskf = (seq_ref[...] > 0).astype(jnp.float32)
    cnt = maskf[0]
    for l in range(1, L):
        cnt = cnt + maskf[l]
    last = jnp.clip(cnt.astype(jnp.int32) - 1, 0, L - 1)

    ht_lo = jnp.zeros((HIDDEN, _BL), jnp.float32)
    ht_hi = jnp.zeros((HIDDEN, _BL), jnp.float32)
    for l in range(L):
        sl = (last == l).astype(jnp.float32)
        ht_lo = ht_lo + sl * s_lo[l]
        ht_hi = ht_hi + sl * s_hi[l]

    w1t = w1_ref[...]
    w2t = w2_ref[...]
    ab = ab_ref[...]
    avr = av_ref[...]
    ht_full = jnp.concatenate([ht_lo, ht_hi], axis=0)
    h1 = jnp.dot(w1t, ht_full, preferred_element_type=jnp.float32)

    out_lo = jnp.zeros((HIDDEN, _BL), jnp.float32)
    out_hi = jnp.zeros((HIDDEN, _BL), jnp.float32)
    for l in range(L):
        sf = jnp.concatenate([s_lo[l], s_hi[l]], axis=0)
        al = jnp.dot(w2t, sf, preferred_element_type=jnp.float32) + h1 + ab
        al = jax.nn.sigmoid(al)
        alpha = jnp.dot(avr, al, preferred_element_type=jnp.float32)
        alpha = alpha * maskf[l]
        out_lo = out_lo + alpha * s_lo[l]
        out_hi = out_hi + alpha * s_hi[l]

    on2 = (jnp.sum(out_lo * out_lo, axis=0, keepdims=True)
           + jnp.sum(out_hi * out_hi, axis=0, keepdims=True))
    oinv = 1.0 / jnp.maximum(jnp.sqrt(on2), 1e-12)
    lo_ref[...] = out_lo * oinv
    hi_ref[...] = out_hi * oinv


def _frontend(nh_t, par_t, adj_t, ai_t, ac_t, seq_t, w1t, w2t, ab, av):
    nb = B // _BL
    bs = [
        pl.BlockSpec((N_NODES, 2 * HIDDEN, _BL), lambda i: (0, 0, i)),
        pl.BlockSpec((N_NODES, 1, _BL), lambda i: (0, 0, i)),
        pl.BlockSpec((N_NODES, N_EDGES, _BL), lambda i: (0, 0, i)),
        pl.BlockSpec((L, 1, _BL), lambda i: (0, 0, i)),
        pl.BlockSpec((L, 1, _BL), lambda i: (0, 0, i)),
        pl.BlockSpec((L, 1, _BL), lambda i: (0, 0, i)),
        pl.BlockSpec((2 * HIDDEN, 2 * HIDDEN), lambda i: (0, 0)),
        pl.BlockSpec((2 * HIDDEN, 2 * HIDDEN), lambda i: (0, 0)),
        pl.BlockSpec((2 * HIDDEN, 1), lambda i: (0, 0)),
        pl.BlockSpec((1, 2 * HIDDEN), lambda i: (0, 0)),
    ]
    out_shape = [
        jax.ShapeDtypeStruct((HIDDEN, B), jnp.float32),
        jax.ShapeDtypeStruct((HIDDEN, B), jnp.float32),
    ]
    out_bs = [
        pl.BlockSpec((HIDDEN, _BL), lambda i: (0, i)),
        pl.BlockSpec((HIDDEN, _BL), lambda i: (0, i)),
    ]
    return pl.pallas_call(
        _frontend_body, grid=(nb,), in_specs=bs, out_specs=out_bs,
        out_shape=out_shape,
    )(nh_t, par_t, adj_t, ai_t, ac_t, seq_t, w1t, w2t, ab, av)


_CPAD = 1024


def _scores_body(lo_ref, hi_ref, emb_ref, cate_ref, ctbl_ref, out_ref):
    cb = cate_ref[...].reshape(1, _VB) - ITEM_NUM
    biota = lax.broadcasted_iota(jnp.int32, (128, _VB), 0)
    ctblt = ctbl_ref[...]
    chalf = None
    for j in range(_CPAD // 128):
        oh = (biota + (j * 128) == cb).astype(jnp.float32)
        p = jnp.dot(ctblt[:, j * 128:(j + 1) * 128], oh,
                    preferred_element_type=jnp.float32)
        chalf = p if chalf is None else chalf + p

    eb = emb_ref[...]
    ones = jnp.ones((1, HIDDEN), jnp.float32)
    n2 = (lax.dot_general(ones, eb * eb,
                          (((1,), (1,)), ((), ())),
                          preferred_element_type=jnp.float32)
          + jnp.dot(ones, chalf * chalf,
                    preferred_element_type=jnp.float32))
    scale = 16.0 / jnp.maximum(jnp.sqrt(n2), 1e-12)

    s = (lax.dot_general(lo_ref[...], eb, (((1,), (1,)), ((), ())),
                         preferred_element_type=jnp.float32)
         + jnp.dot(hi_ref[...], chalf, preferred_element_type=jnp.float32))
    out_ref[...] = s * scale


def _scores(seq_lo, seq_hi, emb_items, cates3, ctblt):
    nv = _NVB
    bs = [
        pl.BlockSpec((B, HIDDEN), lambda v: (0, 0)),
        pl.BlockSpec((B, HIDDEN), lambda v: (0, 0)),
        pl.BlockSpec((_VB, HIDDEN), lambda v: (v, 0)),
        pl.BlockSpec((1, 1, _VB), lambda v: (v, 0, 0)),
        pl.BlockSpec((HIDDEN, _CPAD), lambda v: (0, 0)),
    ]
    return pl.pallas_call(
        _scores_body, grid=(nv,), in_specs=bs,
        out_specs=pl.BlockSpec((B, _VB), lambda v: (0, v)),
        out_shape=jax.ShapeDtypeStruct((B, ITEM_NUM), jnp.float32),
    )(seq_lo, seq_hi, emb_items, cates3, ctblt)


def kernel(item_seq, label, nodes, hn_adj, alias_item, alias_cate, emb,
           item_cates, W1, W2, att_b, att_v):
    del label
    emb = emb.astype(jnp.float32)
    nodes_t = nodes.astype(jnp.int32).T
    emb_pairs = emb.reshape((ITEM_NUM + CATE_NUM) // 2, 2 * HIDDEN)
    nh = _sc_gather(emb_pairs, (nodes_t >> 1).reshape(-1))
    nh_t = nh.reshape(N_NODES, B, 2 * HIDDEN).transpose(0, 2, 1)
    par_t = (nodes_t & 1).astype(jnp.float32).reshape(N_NODES, 1, B)

    lo_t, hi_t = _frontend(
        nh_t, par_t, hn_adj.astype(jnp.float32).transpose(1, 2, 0),
        alias_item.astype(jnp.int32).T.reshape(L, 1, B),
        alias_cate.astype(jnp.int32).T.reshape(L, 1, B),
        item_seq.astype(jnp.int32).T.reshape(L, 1, B),
        W1.astype(jnp.float32).T, W2.astype(jnp.float32).T,
        att_b.astype(jnp.float32).reshape(-1, 1),
        att_v.astype(jnp.float32).reshape(1, -1))
    seq_lo, seq_hi = lo_t.T, hi_t.T

    emb_items = emb[:ITEM_NUM]
    ctblt = jnp.pad(emb[ITEM_NUM:].T, ((0, 0), (0, _CPAD - CATE_NUM)))
    cates_pad = jnp.pad(item_cates.astype(jnp.int32),
                        (0, _NVB * _VB - ITEM_NUM),
                        constant_values=ITEM_NUM)
    cates3 = cates_pad.reshape(_NVB, 1, _VB)
    return _scores(seq_lo, seq_hi, emb_items, cates3, ctblt)


# speedup vs baseline: 1.8651x; 1.0225x over previous
import functools

import jax
import jax.numpy as jnp
from jax import lax
from jax.experimental import pallas as pl
from jax.experimental.pallas import tpu as pltpu
from jax.experimental.pallas import tpu_sc as plsc

ITEM_NUM = 100000
CATE_NUM = 1000
HIDDEN = 64
B = 1024
L = 50
N_NODES = 30
N_EDGES = 20

_BL = 128
_VB = 2048
_NVB = -(-ITEM_NUM // _VB)


def _sc_gather(emb, idx):
    info = plsc.get_sparse_core_info()
    nw = info.num_cores * info.num_subcores
    n, d = idx.shape[0], emb.shape[1]
    bpw = n // nw
    mesh = plsc.VectorSubcoreMesh(core_axis_name="c", subcore_axis_name="s")

    @functools.partial(
        pl.kernel,
        mesh=mesh,
        out_type=jax.ShapeDtypeStruct((n, d), jnp.float32),
        scratch_types=[
            pltpu.VMEM((bpw,), jnp.int32),
            pltpu.VMEM((bpw, d), jnp.float32),
            pltpu.SemaphoreType.DMA,
        ],
    )
    def gather_k(emb_hbm, idx_hbm, out_hbm, idx_v, rows_v, sem):
        wid = lax.axis_index("s") * info.num_cores + lax.axis_index("c")
        base = wid * bpw
        pltpu.sync_copy(idx_hbm.at[pl.ds(base, bpw)], idx_v)
        pltpu.async_copy(emb_hbm.at[idx_v], rows_v, sem).wait()
        pltpu.sync_copy(rows_v, out_hbm.at[pl.ds(base, bpw)])

    return gather_k(emb, idx)


def _frontend_body(nh_ref, par_ref, adj_ref, ai_ref, ac_ref, seq_ref,
                   w1_ref, w2_ref, ab_ref, av_ref, lo_ref, hi_ref):
    i2h = lax.broadcasted_iota(jnp.int32, (2 * HIDDEN, 2 * HIDDEN), 0)
    j2h = lax.broadcasted_iota(jnp.int32, (2 * HIDDEN, 2 * HIDDEN), 1)
    id2h = (i2h == j2h).astype(jnp.float32)
    im = lax.broadcasted_iota(jnp.int32, (N_EDGES, N_EDGES), 0)
    jm = lax.broadcasted_iota(jnp.int32, (N_EDGES, N_EDGES), 1)
    idm = (im == jm).astype(jnp.float32)
    _nt = (((1,), (1,)), ((), ()))

    graw = nh_ref[...]
    gadj = adj_ref[...]
    par = par_ref[...]
    x = []
    adj = []
    for n in range(N_NODES):
        xt = lax.dot_general(id2h, graw[n], _nt,
                             preferred_element_type=jnp.float32)
        xn = jnp.where(par[n] == 1.0, xt[HIDDEN:, :], xt[:HIDDEN, :])
        n2 = jnp.sum(xn * xn, axis=0, keepdims=True)
        x.append(xn / jnp.maximum(jnp.sqrt(n2), 1e-12))
        adj.append(lax.dot_general(idm, gadj[n], _nt,
                                   preferred_element_type=jnp.float32))

    dv = [jnp.maximum(jnp.sum(a, axis=0, keepdims=True), 1e-6)
          for a in adj]
    de = adj[0]
    for n in range(1, N_NODES):
        de = de + adj[n]
    de = jnp.maximum(de, 1e-6)

    edges = []
    for m in range(N_EDGES):
        e = adj[0][m:m + 1, :] * x[0]
        for n in range(1, N_NODES):
            e = e + adj[n][m:m + 1, :] * x[n]
        edges.append(e / de[m:m + 1, :])

    ai = ai_ref[...]
    ac = ac_ref[...]
    ha_i = jnp.zeros((L, N_EDGES, _BL), jnp.float32)
    ha_c = jnp.zeros((L, N_EDGES, _BL), jnp.float32)
    dva_i = jnp.zeros((L, 1, _BL), jnp.float32)
    dva_c = jnp.zeros((L, 1, _BL), jnp.float32)
    for n in range(N_NODES):
        mi = (ai == n).astype(jnp.float32)
        mc = (ac == n).astype(jnp.float32)
        an = adj[n][None, :, :]
        ha_i = ha_i + mi * an
        ha_c = ha_c + mc * an
        dva_i = dva_i + mi * dv[n][None, :, :]
        dva_c = dva_c + mc * dv[n][None, :, :]

    s_lo = jnp.zeros((L, HIDDEN, _BL), jnp.float32)
    s_hi = jnp.zeros((L, HIDDEN, _BL), jnp.float32)
    for m in range(N_EDGES):
        em = edges[m][None, :, :]
        s_lo = s_lo + ha_i[:, m:m + 1, :] * em
        s_hi = s_hi + ha_c[:, m:m + 1, :] * em
    s_lo = s_lo / dva_i
    s_hi = s_hi / dva_c

    nrm = jnp.sqrt(jnp.sum(s_lo * s_lo, axis=1, keepdims=True)
                   + jnp.sum(s_hi * s_hi, axis=1, keepdims=True))
    inv = 1.0 / jnp.maximum(nrm, 1e-12)
    s_lo = s_lo * inv
    s_hi = s_hi * inv

    maskf = (seq_ref[...] > 0).astype(jnp.float32)
    cnt = maskf[0]
    for l in range(1, L):
        cnt = cnt + maskf[l]
    last = jnp.clip(cnt.astype(jnp.int32) - 1, 0, L - 1)

    ht_lo = jnp.zeros((HIDDEN, _BL), jnp.float32)
    ht_hi = jnp.zeros((HIDDEN, _BL), jnp.float32)
    for l in range(L):
        sl = (last == l).astype(jnp.float32)
        ht_lo = ht_lo + sl * s_lo[l]
        ht_hi = ht_hi + sl * s_hi[l]

    w1t = w1_ref[...]
    w2t = w2_ref[...]
    ab = ab_ref[...]
    avr = av_ref[...]
    ht_full = jnp.concatenate([ht_lo, ht_hi], axis=0)
    h1 = jnp.dot(w1t, ht_full, preferred_element_type=jnp.float32)

    out_lo = jnp.zeros((HIDDEN, _BL), jnp.float32)
    out_hi = jnp.zeros((HIDDEN, _BL), jnp.float32)
    for l in range(L):
        sf = jnp.concatenate([s_lo[l], s_hi[l]], axis=0)
        al = jnp.dot(w2t, sf, preferred_element_type=jnp.float32) + h1 + ab
        al = jax.nn.sigmoid(al)
        alpha = jnp.dot(avr, al, preferred_element_type=jnp.float32)
        alpha = alpha * maskf[l]
        out_lo = out_lo + alpha * s_lo[l]
        out_hi = out_hi + alpha * s_hi[l]

    on2 = (jnp.sum(out_lo * out_lo, axis=0, keepdims=True)
           + jnp.sum(out_hi * out_hi, axis=0, keepdims=True))
    oinv = 1.0 / jnp.maximum(jnp.sqrt(on2), 1e-12)
    lo_ref[...] = out_lo * oinv
    hi_ref[...] = out_hi * oinv


def _frontend(nh_t, par_t, adj_t, ai_t, ac_t, seq_t, w1t, w2t, ab, av):
    nb = B // _BL
    bs = [
        pl.BlockSpec((N_NODES, _BL, 2 * HIDDEN), lambda i: (0, i, 0)),
        pl.BlockSpec((N_NODES, 1, _BL), lambda i: (0, 0, i)),
        pl.BlockSpec((N_NODES, _BL, N_EDGES), lambda i: (0, i, 0)),
        pl.BlockSpec((L, 1, _BL), lambda i: (0, 0, i)),
        pl.BlockSpec((L, 1, _BL), lambda i: (0, 0, i)),
        pl.BlockSpec((L, 1, _BL), lambda i: (0, 0, i)),
        pl.BlockSpec((2 * HIDDEN, 2 * HIDDEN), lambda i: (0, 0)),
        pl.BlockSpec((2 * HIDDEN, 2 * HIDDEN), lambda i: (0, 0)),
        pl.BlockSpec((2 * HIDDEN, 1), lambda i: (0, 0)),
        pl.BlockSpec((1, 2 * HIDDEN), lambda i: (0, 0)),
    ]
    out_shape = [
        jax.ShapeDtypeStruct((HIDDEN, B), jnp.float32),
        jax.ShapeDtypeStruct((HIDDEN, B), jnp.float32),
    ]
    out_bs = [
        pl.BlockSpec((HIDDEN, _BL), lambda i: (0, i)),
        pl.BlockSpec((HIDDEN, _BL), lambda i: (0, i)),
    ]
    return pl.pallas_call(
        _frontend_body, grid=(nb,), in_specs=bs, out_specs=out_bs,
        out_shape=out_shape,
    )(nh_t, par_t, adj_t, ai_t, ac_t, seq_t, w1t, w2t, ab, av)


_CPAD = 1024


def _scores_body(lo_ref, hi_ref, emb_ref, cate_ref, ctbl_ref, out_ref):
    cb = cate_ref[...].reshape(1, _VB) - ITEM_NUM
    biota = lax.broadcasted_iota(jnp.int32, (128, _VB), 0)
    ctblt = ctbl_ref[...]
    chalf = None
    for j in range(_CPAD // 128):
        oh = (biota + (j * 128) == cb).astype(jnp.float32)
        p = jnp.dot(ctblt[:, j * 128:(j + 1) * 128], oh,
                    preferred_element_type=jnp.float32)
        chalf = p if chalf is None else chalf + p

    eb = emb_ref[...]
    ones = jnp.ones((1, HIDDEN), jnp.float32)
    n2 = (lax.dot_general(ones, eb * eb,
                          (((1,), (1,)), ((), ())),
                          preferred_element_type=jnp.float32)
          + jnp.dot(ones, chalf * chalf,
                    preferred_element_type=jnp.float32))
    scale = 16.0 / jnp.maximum(jnp.sqrt(n2), 1e-12)

    s = (lax.dot_general(lo_ref[...], eb, (((1,), (1,)), ((), ())),
                         preferred_element_type=jnp.float32)
         + jnp.dot(hi_ref[...], chalf, preferred_element_type=jnp.float32))
    out_ref[...] = s * scale


def _scores(seq_lo, seq_hi, emb_items, cates3, ctblt):
    nv = _NVB
    bs = [
        pl.BlockSpec((B, HIDDEN), lambda v: (0, 0)),
        pl.BlockSpec((B, HIDDEN), lambda v: (0, 0)),
        pl.BlockSpec((_VB, HIDDEN), lambda v: (v, 0)),
        pl.BlockSpec((1, 1, _VB), lambda v: (v, 0, 0)),
        pl.BlockSpec((HIDDEN, _CPAD), lambda v: (0, 0)),
    ]
    return pl.pallas_call(
        _scores_body, grid=(nv,), in_specs=bs,
        out_specs=pl.BlockSpec((B, _VB), lambda v: (0, v)),
        out_shape=jax.ShapeDtypeStruct((B, ITEM_NUM), jnp.float32),
    )(seq_lo, seq_hi, emb_items, cates3, ctblt)


def kernel(item_seq, label, nodes, hn_adj, alias_item, alias_cate, emb,
           item_cates, W1, W2, att_b, att_v):
    del label
    emb = emb.astype(jnp.float32)
    nodes_t = nodes.astype(jnp.int32).T
    emb_pairs = emb.reshape((ITEM_NUM + CATE_NUM) // 2, 2 * HIDDEN)
    nh = _sc_gather(emb_pairs, (nodes_t >> 1).reshape(-1))
    nh3 = nh.reshape(N_NODES, B, 2 * HIDDEN)
    par_t = (nodes_t & 1).astype(jnp.float32).reshape(N_NODES, 1, B)

    lo_t, hi_t = _frontend(
        nh3, par_t, hn_adj.astype(jnp.float32).transpose(1, 0, 2),
        alias_item.astype(jnp.int32).T.reshape(L, 1, B),
        alias_cate.astype(jnp.int32).T.reshape(L, 1, B),
        item_seq.astype(jnp.int32).T.reshape(L, 1, B),
        W1.astype(jnp.float32).T, W2.astype(jnp.float32).T,
        att_b.astype(jnp.float32).reshape(-1, 1),
        att_v.astype(jnp.float32).reshape(1, -1))
    seq_lo, seq_hi = lo_t.T, hi_t.T

    ctblt = jnp.pad(emb[ITEM_NUM:].T, ((0, 0), (0, _CPAD - CATE_NUM)))
    cates_pad = jnp.pad(item_cates.astype(jnp.int32),
                        (0, _NVB * _VB - ITEM_NUM),
                        constant_values=ITEM_NUM)
    cates3 = cates_pad.reshape(_NVB, 1, _VB)
    return _scores(seq_lo, seq_hi, emb, cates3, ctblt)
